# baseline (speedup 1.0000x reference)
import functools

import jax
import jax.numpy as jnp
from jax import lax
from jax.experimental import pallas as pl
from jax.experimental.pallas import tpu as pltpu
from jax.experimental.pallas import tpu_sc as plsc

N = 10000
E = 320000
D = 128
DH = 256
NC = 2
NS = 16
NW = NC * NS
EPT = E // NW
C = 80
NCHUNK = EPT // C
NPAD = 10240
RPS = NPAD // NS
RB = 2000
GRID = N // RB
RW = RPS // C
SLAB = 8
NCHUNKP = 128
NSLAB = NCHUNKP // SLAB


def _sc_agg_body(table, srcs3, dsts3, z128,
                 part_a, part_b,
                 acc, idx_s, idx_d, rows0, rows1, sem_i, sg0, sg1):
    c = lax.axis_index("c")
    s = lax.axis_index("s")
    wid = c * NS + s
    row0 = s * RPS
    pltpu.sync_copy(z128, rows0)
    for k in range(RW):
        pltpu.sync_copy(rows0, acc.at[pl.ds(row0 + k * C, C)])
    plsc.subcore_barrier()

    rows = [rows0, rows1]
    sg = [sg0, sg1]

    def slab(g, carry):
        g0 = pl.multiple_of(g * SLAB, SLAB)
        pltpu.sync_copy(srcs3.at[wid].at[pl.ds(g0, SLAB)], idx_s)
        pltpu.sync_copy(dsts3.at[wid].at[pl.ds(g0, SLAB)], idx_d)
        pltpu.async_copy(table.at[idx_s.at[0]], rows0, sg0)
        for j in range(1, SLAB):
            pltpu.async_copy(table.at[idx_s.at[j]], rows[j % 2], sg[j % 2])
            pltpu.make_async_copy(
                table.at[idx_s.at[j - 1]], rows[(j - 1) % 2],
                sg[(j - 1) % 2]).wait()
            pltpu.sync_copy(rows[(j - 1) % 2], acc.at[idx_d.at[j - 1]],
                            add=True)
        pltpu.make_async_copy(
            table.at[idx_s.at[SLAB - 1]], rows[(SLAB - 1) % 2],
            sg[(SLAB - 1) % 2]).wait()
        pltpu.sync_copy(rows[(SLAB - 1) % 2], acc.at[idx_d.at[SLAB - 1]],
                        add=True)
        return carry

    lax.fori_loop(0, NSLAB, slab, 0)
    plsc.subcore_barrier()

    part = [part_a, part_b]
    for ci in range(NC):
        @pl.when(c == ci)
        def _(ci=ci):
            for k in range(RW):
                r0 = row0 + k * C
                pltpu.sync_copy(acc.at[pl.ds(r0, C)], rows0)
                pltpu.sync_copy(rows0, part[ci].at[pl.ds(r0, C)])


@functools.cache
def _sc_kernel():
    mesh = plsc.VectorSubcoreMesh(core_axis_name="c", subcore_axis_name="s")
    return pl.kernel(
        _sc_agg_body,
        out_type=(
            jax.ShapeDtypeStruct((NPAD, D), jnp.float32),
            jax.ShapeDtypeStruct((NPAD, D), jnp.float32),
        ),
        mesh=mesh,
        scratch_types=[
            pltpu.VMEM_SHARED((NPAD, D), jnp.float32),
            pltpu.VMEM((SLAB, C), jnp.int32),
            pltpu.VMEM((SLAB, C), jnp.int32),
            pltpu.VMEM((C, D), jnp.float32),
            pltpu.VMEM((C, D), jnp.float32),
            pltpu.SemaphoreType.DMA,
            pltpu.SemaphoreType.DMA,
            pltpu.SemaphoreType.DMA,
        ],
        name="sc_gather_segsum",
    )


def _sc_agg(*args):
    return _sc_kernel()(*args)


def _relu_ln(v, w, b):
    mu = jnp.mean(v, axis=-1, keepdims=True)
    var = jnp.mean((v - mu) ** 2, axis=-1, keepdims=True)
    return jax.nn.relu((v - mu) * lax.rsqrt(var + 1e-5) * w + b)


def _dot_t(a, w):
    return lax.dot_general(a, w, (((1,), (1,)), ((), ())),
                           preferred_element_type=jnp.float32)


def _t0_body(x_ref, w1_ref, b1_ref, lnw_ref, lnb_ref, x0_ref, x1_ref, t0_ref):
    h = _dot_t(x_ref[...], w1_ref[...]) + b1_ref[...]
    x0 = h[:, :D]
    x1 = h[:, D:]
    x0_ref[...] = x0
    x1_ref[...] = x1
    t0_ref[...] = _relu_ln(x1, lnw_ref[...], lnb_ref[...])


def _tmid_body(res_ref, pa_ref, pb_ref, ca_ref, cb_ref, tab_ref,
               wl_ref, bl_ref, wr_ref, lnw_ref, lnb_ref, y_ref, tn_ref):
    cnt = ca_ref[:, 0] + cb_ref[:, 0]
    inv = 1.0 / jnp.maximum(cnt, 1.0)
    agg = (pa_ref[...] + pb_ref[...]) * inv[:, None]
    y = (res_ref[...] + _dot_t(agg, wl_ref[...]) + bl_ref[...]
         + _dot_t(tab_ref[...], wr_ref[...]))
    y_ref[...] = y
    tn_ref[...] = _relu_ln(y, lnw_ref[...], lnb_ref[...])


def _tfin_body(res_ref, pa_ref, pb_ref, ca_ref, cb_ref, tab_ref,
               wl_ref, bl_ref, wr_ref, y2_ref, nw_ref, nb_ref,
               w2_ref, b2_ref, out_ref):
    cnt = ca_ref[:, 0] + cb_ref[:, 0]
    inv = 1.0 / jnp.maximum(cnt, 1.0)
    agg = (pa_ref[...] + pb_ref[...]) * inv[:, None]
    y3 = (res_ref[...] + _dot_t(agg, wl_ref[...]) + bl_ref[...]
          + _dot_t(tab_ref[...], wr_ref[...]))
    h = jnp.concatenate([y2_ref[...], y3], axis=1)
    hn = _relu_ln(h, nw_ref[...], nb_ref[...])
    out_ref[...] = _dot_t(hn, w2_ref[...]) + b2_ref[...]


def _rows(shape):
    return pl.BlockSpec((RB,) + shape[1:], lambda i: (i,) + (0,) * (len(shape) - 1))


def _full(shape):
    return pl.BlockSpec(shape, lambda i: (0,) * len(shape))


_f32 = jnp.float32


def _t0_call(x, w1, b1, lnw, lnb):
    return pl.pallas_call(
        _t0_body,
        grid=(GRID,),
        in_specs=[_rows((N, D)), _full((DH, D)), _full((1, DH)),
                  _full((1, D)), _full((1, D))],
        out_specs=(_rows((N, D)), _rows((N, D)), _rows((N, D))),
        out_shape=(jax.ShapeDtypeStruct((N, D), _f32),) * 3,
        name="tc_lin1_ln",
    )(x, w1, b1, lnw, lnb)


def _tmid_call(res, pa, pb, ca, cb, tab, wl, bl, wr, lnw, lnb):
    return pl.pallas_call(
        _tmid_body,
        grid=(GRID,),
        in_specs=[_rows((N, D)), _rows((N, D)), _rows((N, D)),
                  _rows((N, D)), _rows((N, D)), _rows((N, D)),
                  _full((D, D)), _full((1, D)), _full((D, D)),
                  _full((1, D)), _full((1, D))],
        out_specs=(_rows((N, D)), _rows((N, D))),
        out_shape=(jax.ShapeDtypeStruct((N, D), _f32),) * 2,
        name="tc_block_post",
    )(res, pa, pb, ca, cb, tab, wl, bl, wr, lnw, lnb)


def _tfin_call(res, pa, pb, ca, cb, tab, wl, bl, wr, y2, nw, nb, w2, b2):
    return pl.pallas_call(
        _tfin_body,
        grid=(GRID,),
        in_specs=[_rows((N, D)), _rows((N, D)), _rows((N, D)),
                  _rows((N, D)), _rows((N, D)), _rows((N, D)),
                  _full((D, D)), _full((1, D)), _full((D, D)),
                  _rows((N, D)), _full((1, DH)), _full((1, DH)),
                  _full((D, DH)), _full((1, D))],
        out_specs=_rows((N, D)),
        out_shape=jax.ShapeDtypeStruct((N, D), _f32),
        name="tc_final",
    )(res, pa, pb, ca, cb, tab, wl, bl, wr, y2, nw, nb, w2, b2)


def kernel(x, edge_index, lin1_w, lin1_b,
           l0g0_ln_w, l0g0_ln_b, l0g0_wl, l0g0_bl, l0g0_wr,
           l0g1_ln_w, l0g1_ln_b, l0g1_wl, l0g1_bl, l0g1_wr,
           l1g0_ln_w, l1g0_ln_b, l1g0_wl, l1g0_bl, l1g0_wr,
           l1g1_ln_w, l1g1_ln_b, l1g1_wl, l1g1_bl, l1g1_wr,
           norm_w, norm_b, lin2_w, lin2_b):
    pad = ((0, 0), (0, NCHUNKP - NCHUNK), (0, 0))
    src = jnp.pad(edge_index[0].astype(jnp.int32).reshape(NW, NCHUNK, C), pad)
    dst = jnp.pad(edge_index[1].astype(jnp.int32).reshape(NW, NCHUNK, C), pad,
                  constant_values=N)
    z128 = jnp.zeros((C, D), _f32)
    ones_tab = jnp.ones((N, D), _f32)

    r2 = lambda v: v.reshape(1, -1)

    ca, cb = _sc_agg(ones_tab, src, dst, z128)

    x0, x1, t0 = _t0_call(x, lin1_w, r2(lin1_b), r2(l0g0_ln_w), r2(l0g0_ln_b))

    pa, pb = _sc_agg(t0, src, dst, z128)
    y0, t1 = _tmid_call(x0, pa, pb, ca, cb, t0,
                        l0g0_wl, r2(l0g0_bl), l0g0_wr,
                        r2(l0g1_ln_w), r2(l0g1_ln_b))

    pa, pb = _sc_agg(t1, src, dst, z128)
    y1, t2 = _tmid_call(x1, pa, pb, ca, cb, t1,
                        l0g1_wl, r2(l0g1_bl), l0g1_wr,
                        r2(l1g0_ln_w), r2(l1g0_ln_b))

    pa, pb = _sc_agg(t2, src, dst, z128)
    y2, t3 = _tmid_call(y0, pa, pb, ca, cb, t2,
                        l1g0_wl, r2(l1g0_bl), l1g0_wr,
                        r2(l1g1_ln_w), r2(l1g1_ln_b))

    pa, pb = _sc_agg(t3, src, dst, z128)
    out = _tfin_call(y1, pa, pb, ca, cb, t3,
                     l1g1_wl, r2(l1g1_bl), l1g1_wr, y2,
                     r2(norm_w), r2(norm_b), lin2_w, r2(lin2_b))
    return out


# speedup vs baseline: 2.2367x; 2.2367x over previous
import functools

import jax
import jax.numpy as jnp
from jax import lax
from jax.experimental import pallas as pl
from jax.experimental.pallas import tpu as pltpu
from jax.experimental.pallas import tpu_sc as plsc

N = 10000
E = 320000
D = 128
DH = 256
NC = 2
NS = 16
NW = NC * NS
EPT = E // NW
C = 80
NCHUNK = EPT // C
NPAD = 10240
RPS = NPAD // NS
RB = 2000
GRID = N // RB
RW = RPS // C


def _sc_agg_body(table, srcs, dsts, z128,
                 part_a, part_b,
                 acc, sidx_a, didx_a, sidx_b, didx_b,
                 rows_a, rows_b, sg_a, sg_b):
    c = lax.axis_index("c")
    s = lax.axis_index("s")
    wid = c * NS + s
    row0 = s * RPS
    pltpu.sync_copy(z128, rows_a)
    for k in range(RW):
        pltpu.sync_copy(rows_a, acc.at[pl.ds(row0 + k * C, C)])
    plsc.subcore_barrier()

    base = wid * EPT

    def load_idx(i, sidx, didx):
        off = base + i * C
        pltpu.sync_copy(srcs.at[pl.ds(off, C)], sidx)
        pltpu.sync_copy(dsts.at[pl.ds(off, C)], didx)

    load_idx(0, sidx_a, didx_a)
    pltpu.async_copy(table.at[sidx_a], rows_a, sg_a)

    def pair(k, carry):
        i0 = 2 * k
        load_idx(i0 + 1, sidx_b, didx_b)
        pltpu.async_copy(table.at[sidx_b], rows_b, sg_b)
        pltpu.make_async_copy(table.at[sidx_a], rows_a, sg_a).wait()
        pltpu.sync_copy(rows_a, acc.at[didx_a], add=True)
        load_idx(i0 + 2, sidx_a, didx_a)
        pltpu.async_copy(table.at[sidx_a], rows_a, sg_a)
        pltpu.make_async_copy(table.at[sidx_b], rows_b, sg_b).wait()
        pltpu.sync_copy(rows_b, acc.at[didx_b], add=True)
        return carry

    lax.fori_loop(0, (NCHUNK - 1) // 2, pair, 0)
    pltpu.make_async_copy(table.at[sidx_a], rows_a, sg_a).wait()
    pltpu.sync_copy(rows_a, acc.at[didx_a], add=True)
    plsc.subcore_barrier()

    part = [part_a, part_b]
    for ci in range(NC):
        @pl.when(c == ci)
        def _(ci=ci):
            for k in range(RW):
                r0 = row0 + k * C
                pltpu.sync_copy(acc.at[pl.ds(r0, C)], rows_a)
                pltpu.sync_copy(rows_a, part[ci].at[pl.ds(r0, C)])


@functools.cache
def _sc_kernel():
    mesh = plsc.VectorSubcoreMesh(core_axis_name="c", subcore_axis_name="s")
    return pl.kernel(
        _sc_agg_body,
        out_type=(
            jax.ShapeDtypeStruct((NPAD, D), jnp.float32),
            jax.ShapeDtypeStruct((NPAD, D), jnp.float32),
        ),
        mesh=mesh,
        scratch_types=[
            pltpu.VMEM_SHARED((NPAD, D), jnp.float32),
            pltpu.VMEM((C,), jnp.int32),
            pltpu.VMEM((C,), jnp.int32),
            pltpu.VMEM((C,), jnp.int32),
            pltpu.VMEM((C,), jnp.int32),
            pltpu.VMEM((C, D), jnp.float32),
            pltpu.VMEM((C, D), jnp.float32),
            pltpu.SemaphoreType.DMA,
            pltpu.SemaphoreType.DMA,
        ],
        name="sc_gather_segsum",
    )


def _sc_agg(*args):
    return _sc_kernel()(*args)


def _relu_ln(v, w, b):
    mu = jnp.mean(v, axis=-1, keepdims=True)
    var = jnp.mean((v - mu) ** 2, axis=-1, keepdims=True)
    return jax.nn.relu((v - mu) * lax.rsqrt(var + 1e-5) * w + b)


def _dot_t(a, w):
    return lax.dot_general(a, w, (((1,), (1,)), ((), ())),
                           preferred_element_type=jnp.float32)


def _t0_body(x_ref, w1_ref, b1_ref, lnw_ref, lnb_ref, x0_ref, x1_ref, t0_ref):
    h = _dot_t(x_ref[...], w1_ref[...]) + b1_ref[...]
    x0 = h[:, :D]
    x1 = h[:, D:]
    x0_ref[...] = x0
    x1_ref[...] = x1
    t0_ref[...] = _relu_ln(x1, lnw_ref[...], lnb_ref[...])


def _tmid_body(res_ref, pa_ref, pb_ref, ca_ref, cb_ref, tab_ref,
               wl_ref, bl_ref, wr_ref, lnw_ref, lnb_ref, y_ref, tn_ref):
    cnt = ca_ref[:, 0] + cb_ref[:, 0]
    inv = 1.0 / jnp.maximum(cnt, 1.0)
    agg = (pa_ref[...] + pb_ref[...]) * inv[:, None]
    y = (res_ref[...] + _dot_t(agg, wl_ref[...]) + bl_ref[...]
         + _dot_t(tab_ref[...], wr_ref[...]))
    y_ref[...] = y
    tn_ref[...] = _relu_ln(y, lnw_ref[...], lnb_ref[...])


def _tfin_body(res_ref, pa_ref, pb_ref, ca_ref, cb_ref, tab_ref,
               wl_ref, bl_ref, wr_ref, y2_ref, nw_ref, nb_ref,
               w2_ref, b2_ref, out_ref):
    cnt = ca_ref[:, 0] + cb_ref[:, 0]
    inv = 1.0 / jnp.maximum(cnt, 1.0)
    agg = (pa_ref[...] + pb_ref[...]) * inv[:, None]
    y3 = (res_ref[...] + _dot_t(agg, wl_ref[...]) + bl_ref[...]
          + _dot_t(tab_ref[...], wr_ref[...]))
    h = jnp.concatenate([y2_ref[...], y3], axis=1)
    hn = _relu_ln(h, nw_ref[...], nb_ref[...])
    out_ref[...] = _dot_t(hn, w2_ref[...]) + b2_ref[...]


def _rows(shape):
    return pl.BlockSpec((RB,) + shape[1:], lambda i: (i,) + (0,) * (len(shape) - 1))


def _full(shape):
    return pl.BlockSpec(shape, lambda i: (0,) * len(shape))


_f32 = jnp.float32


def _t0_call(x, w1, b1, lnw, lnb):
    return pl.pallas_call(
        _t0_body,
        grid=(GRID,),
        in_specs=[_rows((N, D)), _full((DH, D)), _full((1, DH)),
                  _full((1, D)), _full((1, D))],
        out_specs=(_rows((N, D)), _rows((N, D)), _rows((N, D))),
        out_shape=(jax.ShapeDtypeStruct((N, D), _f32),) * 3,
        name="tc_lin1_ln",
    )(x, w1, b1, lnw, lnb)


def _tmid_call(res, pa, pb, ca, cb, tab, wl, bl, wr, lnw, lnb):
    return pl.pallas_call(
        _tmid_body,
        grid=(GRID,),
        in_specs=[_rows((N, D)), _rows((N, D)), _rows((N, D)),
                  _rows((N, D)), _rows((N, D)), _rows((N, D)),
                  _full((D, D)), _full((1, D)), _full((D, D)),
                  _full((1, D)), _full((1, D))],
        out_specs=(_rows((N, D)), _rows((N, D))),
        out_shape=(jax.ShapeDtypeStruct((N, D), _f32),) * 2,
        name="tc_block_post",
    )(res, pa, pb, ca, cb, tab, wl, bl, wr, lnw, lnb)


def _tfin_call(res, pa, pb, ca, cb, tab, wl, bl, wr, y2, nw, nb, w2, b2):
    return pl.pallas_call(
        _tfin_body,
        grid=(GRID,),
        in_specs=[_rows((N, D)), _rows((N, D)), _rows((N, D)),
                  _rows((N, D)), _rows((N, D)), _rows((N, D)),
                  _full((D, D)), _full((1, D)), _full((D, D)),
                  _rows((N, D)), _full((1, DH)), _full((1, DH)),
                  _full((D, DH)), _full((1, D))],
        out_specs=_rows((N, D)),
        out_shape=jax.ShapeDtypeStruct((N, D), _f32),
        name="tc_final",
    )(res, pa, pb, ca, cb, tab, wl, bl, wr, y2, nw, nb, w2, b2)


def kernel(x, edge_index, lin1_w, lin1_b,
           l0g0_ln_w, l0g0_ln_b, l0g0_wl, l0g0_bl, l0g0_wr,
           l0g1_ln_w, l0g1_ln_b, l0g1_wl, l0g1_bl, l0g1_wr,
           l1g0_ln_w, l1g0_ln_b, l1g0_wl, l1g0_bl, l1g0_wr,
           l1g1_ln_w, l1g1_ln_b, l1g1_wl, l1g1_bl, l1g1_wr,
           norm_w, norm_b, lin2_w, lin2_b):
    src = edge_index[0].astype(jnp.int32)
    dst = edge_index[1].astype(jnp.int32)
    z128 = jnp.zeros((C, D), _f32)
    ones_tab = jnp.ones((N, D), _f32)

    r2 = lambda v: v.reshape(1, -1)

    ca, cb = _sc_agg(ones_tab, src, dst, z128)

    x0, x1, t0 = _t0_call(x, lin1_w, r2(lin1_b), r2(l0g0_ln_w), r2(l0g0_ln_b))

    pa, pb = _sc_agg(t0, src, dst, z128)
    y0, t1 = _tmid_call(x0, pa, pb, ca, cb, t0,
                        l0g0_wl, r2(l0g0_bl), l0g0_wr,
                        r2(l0g1_ln_w), r2(l0g1_ln_b))

    pa, pb = _sc_agg(t1, src, dst, z128)
    y1, t2 = _tmid_call(x1, pa, pb, ca, cb, t1,
                        l0g1_wl, r2(l0g1_bl), l0g1_wr,
                        r2(l1g0_ln_w), r2(l1g0_ln_b))

    pa, pb = _sc_agg(t2, src, dst, z128)
    y2, t3 = _tmid_call(y0, pa, pb, ca, cb, t2,
                        l1g0_wl, r2(l1g0_bl), l1g0_wr,
                        r2(l1g1_ln_w), r2(l1g1_ln_b))

    pa, pb = _sc_agg(t3, src, dst, z128)
    out = _tfin_call(y1, pa, pb, ca, cb, t3,
                     l1g1_wl, r2(l1g1_bl), l1g1_wr, y2,
                     r2(norm_w), r2(norm_b), lin2_w, r2(lin2_b))
    return out


# speedup vs baseline: 2.7840x; 1.2447x over previous
import functools

import jax
import jax.numpy as jnp
from jax import lax
from jax.experimental import pallas as pl
from jax.experimental.pallas import tpu as pltpu
from jax.experimental.pallas import tpu_sc as plsc

N = 10000
E = 320000
D = 128
DH = 256
NC = 2
NS = 16
NW = NC * NS
EPT = E // NW
C = 80
NCHUNK = EPT // C
NPAD = 10240
RPS = NPAD // NS
RB = 2000
GRID = N // RB
RW = RPS // C


def _sc_agg_body(table, srcs, dsts, z128,
                 part_a, part_b,
                 acc, sidx_a, didx_a, sidx_b, didx_b,
                 rows_a, rows_b, sg_a, sg_b, si_a, si_b):
    c = lax.axis_index("c")
    s = lax.axis_index("s")
    wid = c * NS + s
    row0 = s * RPS
    pltpu.sync_copy(z128, rows_a)
    for k in range(RW):
        pltpu.sync_copy(rows_a, acc.at[pl.ds(row0 + k * C, C)])
    plsc.subcore_barrier()

    base = wid * EPT

    def load_idx(i, sidx, didx, sem):
        off = base + i * C
        pltpu.async_copy(srcs.at[pl.ds(off, C)], sidx, sem)
        pltpu.async_copy(dsts.at[pl.ds(off, C)], didx, sem)

    def wait_idx(i, sidx, didx, sem):
        off = base + i * C
        pltpu.make_async_copy(srcs.at[pl.ds(off, C)], sidx, sem).wait()
        pltpu.make_async_copy(dsts.at[pl.ds(off, C)], didx, sem).wait()

    load_idx(0, sidx_a, didx_a, si_a)
    wait_idx(0, sidx_a, didx_a, si_a)
    pltpu.async_copy(table.at[sidx_a], rows_a, sg_a)
    load_idx(1, sidx_b, didx_b, si_b)

    def pair(k, carry):
        i0 = 2 * k
        wait_idx(i0 + 1, sidx_b, didx_b, si_b)
        pltpu.async_copy(table.at[sidx_b], rows_b, sg_b)
        pltpu.make_async_copy(table.at[sidx_a], rows_a, sg_a).wait()
        pltpu.sync_copy(rows_a, acc.at[didx_a], add=True)
        load_idx(i0 + 2, sidx_a, didx_a, si_a)
        wait_idx(i0 + 2, sidx_a, didx_a, si_a)
        pltpu.async_copy(table.at[sidx_a], rows_a, sg_a)
        nxt = jnp.minimum(i0 + 3, NCHUNK - 1)
        load_idx(nxt, sidx_b, didx_b, si_b)
        pltpu.make_async_copy(table.at[sidx_b], rows_b, sg_b).wait()
        pltpu.sync_copy(rows_b, acc.at[didx_b], add=True)
        return carry

    lax.fori_loop(0, (NCHUNK - 1) // 2, pair, 0)
    pltpu.make_async_copy(table.at[sidx_a], rows_a, sg_a).wait()
    pltpu.sync_copy(rows_a, acc.at[didx_a], add=True)
    wait_idx(NCHUNK - 1, sidx_b, didx_b, si_b)
    plsc.subcore_barrier()

    part = [part_a, part_b]
    for ci in range(NC):
        @pl.when(c == ci)
        def _(ci=ci):
            for k in range(RW):
                r0 = row0 + k * C
                pltpu.sync_copy(acc.at[pl.ds(r0, C)], rows_a)
                pltpu.sync_copy(rows_a, part[ci].at[pl.ds(r0, C)])


@functools.cache
def _sc_kernel():
    mesh = plsc.VectorSubcoreMesh(core_axis_name="c", subcore_axis_name="s")
    return pl.kernel(
        _sc_agg_body,
        out_type=(
            jax.ShapeDtypeStruct((NPAD, D), jnp.float32),
            jax.ShapeDtypeStruct((NPAD, D), jnp.float32),
        ),
        mesh=mesh,
        scratch_types=[
            pltpu.VMEM_SHARED((NPAD, D), jnp.float32),
            pltpu.VMEM((C,), jnp.int32),
            pltpu.VMEM((C,), jnp.int32),
            pltpu.VMEM((C,), jnp.int32),
            pltpu.VMEM((C,), jnp.int32),
            pltpu.VMEM((C, D), jnp.float32),
            pltpu.VMEM((C, D), jnp.float32),
            pltpu.SemaphoreType.DMA,
            pltpu.SemaphoreType.DMA,
            pltpu.SemaphoreType.DMA,
            pltpu.SemaphoreType.DMA,
        ],
        name="sc_gather_segsum",
    )


def _sc_agg(*args):
    return _sc_kernel()(*args)


def _relu_ln(v, w, b):
    mu = jnp.mean(v, axis=-1, keepdims=True)
    var = jnp.mean((v - mu) ** 2, axis=-1, keepdims=True)
    return jax.nn.relu((v - mu) * lax.rsqrt(var + 1e-5) * w + b)


def _dot_t(a, w):
    return lax.dot_general(a, w, (((1,), (1,)), ((), ())),
                           preferred_element_type=jnp.float32)


def _t0_body(x_ref, w1_ref, b1_ref, lnw_ref, lnb_ref, x0_ref, x1_ref, t0_ref):
    h = _dot_t(x_ref[...], w1_ref[...]) + b1_ref[...]
    x0 = h[:, :D]
    x1 = h[:, D:]
    x0_ref[...] = x0
    x1_ref[...] = x1
    t0_ref[...] = _relu_ln(x1, lnw_ref[...], lnb_ref[...])


def _tmid_body(res_ref, pa_ref, pb_ref, ca_ref, cb_ref, tab_ref,
               wl_ref, bl_ref, wr_ref, lnw_ref, lnb_ref, y_ref, tn_ref):
    cnt = ca_ref[:, 0] + cb_ref[:, 0]
    inv = 1.0 / jnp.maximum(cnt, 1.0)
    agg = (pa_ref[...] + pb_ref[...]) * inv[:, None]
    y = (res_ref[...] + _dot_t(agg, wl_ref[...]) + bl_ref[...]
         + _dot_t(tab_ref[...], wr_ref[...]))
    y_ref[...] = y
    tn_ref[...] = _relu_ln(y, lnw_ref[...], lnb_ref[...])


def _tfin_body(res_ref, pa_ref, pb_ref, ca_ref, cb_ref, tab_ref,
               wl_ref, bl_ref, wr_ref, y2_ref, nw_ref, nb_ref,
               w2_ref, b2_ref, out_ref):
    cnt = ca_ref[:, 0] + cb_ref[:, 0]
    inv = 1.0 / jnp.maximum(cnt, 1.0)
    agg = (pa_ref[...] + pb_ref[...]) * inv[:, None]
    y3 = (res_ref[...] + _dot_t(agg, wl_ref[...]) + bl_ref[...]
          + _dot_t(tab_ref[...], wr_ref[...]))
    h = jnp.concatenate([y2_ref[...], y3], axis=1)
    hn = _relu_ln(h, nw_ref[...], nb_ref[...])
    out_ref[...] = _dot_t(hn, w2_ref[...]) + b2_ref[...]


def _rows(shape):
    return pl.BlockSpec((RB,) + shape[1:], lambda i: (i,) + (0,) * (len(shape) - 1))


def _full(shape):
    return pl.BlockSpec(shape, lambda i: (0,) * len(shape))


_f32 = jnp.float32


def _t0_call(x, w1, b1, lnw, lnb):
    return pl.pallas_call(
        _t0_body,
        grid=(GRID,),
        in_specs=[_rows((N, D)), _full((DH, D)), _full((1, DH)),
                  _full((1, D)), _full((1, D))],
        out_specs=(_rows((N, D)), _rows((N, D)), _rows((N, D))),
        out_shape=(jax.ShapeDtypeStruct((N, D), _f32),) * 3,
        name="tc_lin1_ln",
    )(x, w1, b1, lnw, lnb)


def _tmid_call(res, pa, pb, ca, cb, tab, wl, bl, wr, lnw, lnb):
    return pl.pallas_call(
        _tmid_body,
        grid=(GRID,),
        in_specs=[_rows((N, D)), _rows((N, D)), _rows((N, D)),
                  _rows((N, D)), _rows((N, D)), _rows((N, D)),
                  _full((D, D)), _full((1, D)), _full((D, D)),
                  _full((1, D)), _full((1, D))],
        out_specs=(_rows((N, D)), _rows((N, D))),
        out_shape=(jax.ShapeDtypeStruct((N, D), _f32),) * 2,
        name="tc_block_post",
    )(res, pa, pb, ca, cb, tab, wl, bl, wr, lnw, lnb)


def _tfin_call(res, pa, pb, ca, cb, tab, wl, bl, wr, y2, nw, nb, w2, b2):
    return pl.pallas_call(
        _tfin_body,
        grid=(GRID,),
        in_specs=[_rows((N, D)), _rows((N, D)), _rows((N, D)),
                  _rows((N, D)), _rows((N, D)), _rows((N, D)),
                  _full((D, D)), _full((1, D)), _full((D, D)),
                  _rows((N, D)), _full((1, DH)), _full((1, DH)),
                  _full((D, DH)), _full((1, D))],
        out_specs=_rows((N, D)),
        out_shape=jax.ShapeDtypeStruct((N, D), _f32),
        name="tc_final",
    )(res, pa, pb, ca, cb, tab, wl, bl, wr, y2, nw, nb, w2, b2)


def kernel(x, edge_index, lin1_w, lin1_b,
           l0g0_ln_w, l0g0_ln_b, l0g0_wl, l0g0_bl, l0g0_wr,
           l0g1_ln_w, l0g1_ln_b, l0g1_wl, l0g1_bl, l0g1_wr,
           l1g0_ln_w, l1g0_ln_b, l1g0_wl, l1g0_bl, l1g0_wr,
           l1g1_ln_w, l1g1_ln_b, l1g1_wl, l1g1_bl, l1g1_wr,
           norm_w, norm_b, lin2_w, lin2_b):
    src = edge_index[0].astype(jnp.int32)
    dst = edge_index[1].astype(jnp.int32)
    z128 = jnp.zeros((C, D), _f32)
    ones_tab = jnp.ones((N, D), _f32)

    r2 = lambda v: v.reshape(1, -1)

    ca, cb = _sc_agg(ones_tab, src, dst, z128)

    x0, x1, t0 = _t0_call(x, lin1_w, r2(lin1_b), r2(l0g0_ln_w), r2(l0g0_ln_b))

    pa, pb = _sc_agg(t0, src, dst, z128)
    y0, t1 = _tmid_call(x0, pa, pb, ca, cb, t0,
                        l0g0_wl, r2(l0g0_bl), l0g0_wr,
                        r2(l0g1_ln_w), r2(l0g1_ln_b))

    pa, pb = _sc_agg(t1, src, dst, z128)
    y1, t2 = _tmid_call(x1, pa, pb, ca, cb, t1,
                        l0g1_wl, r2(l0g1_bl), l0g1_wr,
                        r2(l1g0_ln_w), r2(l1g0_ln_b))

    pa, pb = _sc_agg(t2, src, dst, z128)
    y2, t3 = _tmid_call(y0, pa, pb, ca, cb, t2,
                        l1g0_wl, r2(l1g0_bl), l1g0_wr,
                        r2(l1g1_ln_w), r2(l1g1_ln_b))

    pa, pb = _sc_agg(t3, src, dst, z128)
    out = _tfin_call(y1, pa, pb, ca, cb, t3,
                     l1g1_wl, r2(l1g1_bl), l1g1_wr, y2,
                     r2(norm_w), r2(norm_b), lin2_w, r2(lin2_b))
    return out


# speedup vs baseline: 3.0423x; 1.0928x over previous
import functools

import jax
import jax.numpy as jnp
from jax import lax
from jax.experimental import pallas as pl
from jax.experimental.pallas import tpu as pltpu
from jax.experimental.pallas import tpu_sc as plsc

N = 10000
E = 320000
D = 128
DH = 256
NC = 2
NS = 16
NW = NC * NS
EPT = E // NW
C = 80
NCHUNK = EPT // C
NPAD = 10240
RPS = NPAD // NS
RB = 2000
GRID = N // RB
RW = RPS // C


def _sc_agg_body(table, srcs, dsts, z128,
                 part_a, part_b,
                 acc, sidx_a, didx_a, sidx_b, didx_b,
                 rows_a, rows_b, sg_a, sg_b, si_a, si_b):
    c = lax.axis_index("c")
    s = lax.axis_index("s")
    wid = c * NS + s
    row0 = s * RPS
    pltpu.sync_copy(z128, rows_a)
    for k in range(RW):
        pltpu.sync_copy(rows_a, acc.at[pl.ds(row0 + k * C, C)])
    plsc.subcore_barrier()

    base = wid * EPT

    def load_idx(i, sidx, didx, sem):
        off = base + i * C
        pltpu.async_copy(srcs.at[pl.ds(off, C)], sidx, sem)
        pltpu.async_copy(dsts.at[pl.ds(off, C)], didx, sem)

    def wait_idx(i, sidx, didx, sem):
        off = base + i * C
        pltpu.make_async_copy(srcs.at[pl.ds(off, C)], sidx, sem).wait()
        pltpu.make_async_copy(dsts.at[pl.ds(off, C)], didx, sem).wait()

    load_idx(0, sidx_a, didx_a, si_a)
    wait_idx(0, sidx_a, didx_a, si_a)
    pltpu.async_copy(table.at[sidx_a], rows_a, sg_a)
    load_idx(1, sidx_b, didx_b, si_b)

    def pair(k, carry):
        i0 = 2 * k
        wait_idx(i0 + 1, sidx_b, didx_b, si_b)
        pltpu.async_copy(table.at[sidx_b], rows_b, sg_b)
        pltpu.make_async_copy(table.at[sidx_a], rows_a, sg_a).wait()
        pltpu.sync_copy(rows_a, acc.at[didx_a], add=True)
        load_idx(i0 + 2, sidx_a, didx_a, si_a)
        wait_idx(i0 + 2, sidx_a, didx_a, si_a)
        pltpu.async_copy(table.at[sidx_a], rows_a, sg_a)
        nxt = jnp.minimum(i0 + 3, NCHUNK - 1)
        load_idx(nxt, sidx_b, didx_b, si_b)
        pltpu.make_async_copy(table.at[sidx_b], rows_b, sg_b).wait()
        pltpu.sync_copy(rows_b, acc.at[didx_b], add=True)
        return carry

    lax.fori_loop(0, (NCHUNK - 1) // 2, pair, 0)
    pltpu.make_async_copy(table.at[sidx_a], rows_a, sg_a).wait()
    pltpu.sync_copy(rows_a, acc.at[didx_a], add=True)
    wait_idx(NCHUNK - 1, sidx_b, didx_b, si_b)
    plsc.subcore_barrier()

    part = [part_a, part_b]
    for ci in range(NC):
        @pl.when(c == ci)
        def _(ci=ci):
            for k in range(RW):
                r0 = row0 + k * C
                pltpu.sync_copy(acc.at[pl.ds(r0, C)], rows_a)
                pltpu.sync_copy(rows_a, part[ci].at[pl.ds(r0, C)])


def _sc_deg_body(dsts, z128, o128,
                 part_a, part_b,
                 acc, didx_a, didx_b, rows_a, rows_o, si_a, si_b):
    c = lax.axis_index("c")
    s = lax.axis_index("s")
    wid = c * NS + s
    row0 = s * RPS
    pltpu.sync_copy(z128, rows_a)
    for k in range(RW):
        pltpu.sync_copy(rows_a, acc.at[pl.ds(row0 + k * C, C)])
    pltpu.sync_copy(o128, rows_o)
    plsc.subcore_barrier()

    base = wid * EPT

    def load_idx(i, didx, sem):
        pltpu.async_copy(dsts.at[pl.ds(base + i * C, C)], didx, sem)

    def wait_idx(i, didx, sem):
        pltpu.make_async_copy(dsts.at[pl.ds(base + i * C, C)], didx, sem).wait()

    load_idx(0, didx_a, si_a)
    wait_idx(0, didx_a, si_a)
    load_idx(1, didx_b, si_b)

    def pair(k, carry):
        i0 = 2 * k
        pltpu.sync_copy(rows_o, acc.at[didx_a], add=True)
        load_idx(i0 + 2, didx_a, si_a)
        wait_idx(i0 + 1, didx_b, si_b)
        pltpu.sync_copy(rows_o, acc.at[didx_b], add=True)
        load_idx(jnp.minimum(i0 + 3, NCHUNK - 1), didx_b, si_b)
        wait_idx(i0 + 2, didx_a, si_a)
        return carry

    lax.fori_loop(0, (NCHUNK - 1) // 2, pair, 0)
    pltpu.sync_copy(rows_o, acc.at[didx_a], add=True)
    wait_idx(NCHUNK - 1, didx_b, si_b)
    plsc.subcore_barrier()

    part = [part_a, part_b]
    for ci in range(NC):
        @pl.when(c == ci)
        def _(ci=ci):
            for k in range(RW):
                r0 = row0 + k * C
                pltpu.sync_copy(acc.at[pl.ds(r0, C)], rows_a)
                pltpu.sync_copy(rows_a, part[ci].at[pl.ds(r0, C)])


@functools.cache
def _sc_deg_kernel():
    mesh = plsc.VectorSubcoreMesh(core_axis_name="c", subcore_axis_name="s")
    return pl.kernel(
        _sc_deg_body,
        out_type=(
            jax.ShapeDtypeStruct((NPAD, D), jnp.float32),
            jax.ShapeDtypeStruct((NPAD, D), jnp.float32),
        ),
        mesh=mesh,
        scratch_types=[
            pltpu.VMEM_SHARED((NPAD, D), jnp.float32),
            pltpu.VMEM((C,), jnp.int32),
            pltpu.VMEM((C,), jnp.int32),
            pltpu.VMEM((C, D), jnp.float32),
            pltpu.VMEM((C, D), jnp.float32),
            pltpu.SemaphoreType.DMA,
            pltpu.SemaphoreType.DMA,
        ],
        name="sc_degree_segsum",
    )


@functools.cache
def _sc_kernel():
    mesh = plsc.VectorSubcoreMesh(core_axis_name="c", subcore_axis_name="s")
    return pl.kernel(
        _sc_agg_body,
        out_type=(
            jax.ShapeDtypeStruct((NPAD, D), jnp.float32),
            jax.ShapeDtypeStruct((NPAD, D), jnp.float32),
        ),
        mesh=mesh,
        scratch_types=[
            pltpu.VMEM_SHARED((NPAD, D), jnp.float32),
            pltpu.VMEM((C,), jnp.int32),
            pltpu.VMEM((C,), jnp.int32),
            pltpu.VMEM((C,), jnp.int32),
            pltpu.VMEM((C,), jnp.int32),
            pltpu.VMEM((C, D), jnp.float32),
            pltpu.VMEM((C, D), jnp.float32),
            pltpu.SemaphoreType.DMA,
            pltpu.SemaphoreType.DMA,
            pltpu.SemaphoreType.DMA,
            pltpu.SemaphoreType.DMA,
        ],
        name="sc_gather_segsum",
    )


def _sc_agg(*args):
    return _sc_kernel()(*args)


def _relu_ln(v, w, b):
    mu = jnp.mean(v, axis=-1, keepdims=True)
    var = jnp.mean((v - mu) ** 2, axis=-1, keepdims=True)
    return jax.nn.relu((v - mu) * lax.rsqrt(var + 1e-5) * w + b)


def _dot_t(a, w):
    return lax.dot_general(a, w, (((1,), (1,)), ((), ())),
                           preferred_element_type=jnp.float32)


def _t0_body(x_ref, w1_ref, b1_ref, lnw_ref, lnb_ref, x0_ref, x1_ref, t0_ref):
    h = _dot_t(x_ref[...], w1_ref[...]) + b1_ref[...]
    x0 = h[:, :D]
    x1 = h[:, D:]
    x0_ref[...] = x0
    x1_ref[...] = x1
    t0_ref[...] = _relu_ln(x1, lnw_ref[...], lnb_ref[...])


def _tmid_body(res_ref, pa_ref, pb_ref, ca_ref, cb_ref, tab_ref,
               wl_ref, bl_ref, wr_ref, lnw_ref, lnb_ref, y_ref, tn_ref):
    cnt = ca_ref[:, 0] + cb_ref[:, 0]
    inv = 1.0 / jnp.maximum(cnt, 1.0)
    agg = (pa_ref[...] + pb_ref[...]) * inv[:, None]
    y = (res_ref[...] + _dot_t(agg, wl_ref[...]) + bl_ref[...]
         + _dot_t(tab_ref[...], wr_ref[...]))
    y_ref[...] = y
    tn_ref[...] = _relu_ln(y, lnw_ref[...], lnb_ref[...])


def _tfin_body(res_ref, pa_ref, pb_ref, ca_ref, cb_ref, tab_ref,
               wl_ref, bl_ref, wr_ref, y2_ref, nw_ref, nb_ref,
               w2_ref, b2_ref, out_ref):
    cnt = ca_ref[:, 0] + cb_ref[:, 0]
    inv = 1.0 / jnp.maximum(cnt, 1.0)
    agg = (pa_ref[...] + pb_ref[...]) * inv[:, None]
    y3 = (res_ref[...] + _dot_t(agg, wl_ref[...]) + bl_ref[...]
          + _dot_t(tab_ref[...], wr_ref[...]))
    h = jnp.concatenate([y2_ref[...], y3], axis=1)
    hn = _relu_ln(h, nw_ref[...], nb_ref[...])
    out_ref[...] = _dot_t(hn, w2_ref[...]) + b2_ref[...]


def _rows(shape):
    return pl.BlockSpec((RB,) + shape[1:], lambda i: (i,) + (0,) * (len(shape) - 1))


def _full(shape):
    return pl.BlockSpec(shape, lambda i: (0,) * len(shape))


_f32 = jnp.float32


def _t0_call(x, w1, b1, lnw, lnb):
    return pl.pallas_call(
        _t0_body,
        grid=(GRID,),
        in_specs=[_rows((N, D)), _full((DH, D)), _full((1, DH)),
                  _full((1, D)), _full((1, D))],
        out_specs=(_rows((N, D)), _rows((N, D)), _rows((N, D))),
        out_shape=(jax.ShapeDtypeStruct((N, D), _f32),) * 3,
        name="tc_lin1_ln",
    )(x, w1, b1, lnw, lnb)


def _tmid_call(res, pa, pb, ca, cb, tab, wl, bl, wr, lnw, lnb):
    return pl.pallas_call(
        _tmid_body,
        grid=(GRID,),
        in_specs=[_rows((N, D)), _rows((N, D)), _rows((N, D)),
                  _rows((N, D)), _rows((N, D)), _rows((N, D)),
                  _full((D, D)), _full((1, D)), _full((D, D)),
                  _full((1, D)), _full((1, D))],
        out_specs=(_rows((N, D)), _rows((N, D))),
        out_shape=(jax.ShapeDtypeStruct((N, D), _f32),) * 2,
        name="tc_block_post",
    )(res, pa, pb, ca, cb, tab, wl, bl, wr, lnw, lnb)


def _tfin_call(res, pa, pb, ca, cb, tab, wl, bl, wr, y2, nw, nb, w2, b2):
    return pl.pallas_call(
        _tfin_body,
        grid=(GRID,),
        in_specs=[_rows((N, D)), _rows((N, D)), _rows((N, D)),
                  _rows((N, D)), _rows((N, D)), _rows((N, D)),
                  _full((D, D)), _full((1, D)), _full((D, D)),
                  _rows((N, D)), _full((1, DH)), _full((1, DH)),
                  _full((D, DH)), _full((1, D))],
        out_specs=_rows((N, D)),
        out_shape=jax.ShapeDtypeStruct((N, D), _f32),
        name="tc_final",
    )(res, pa, pb, ca, cb, tab, wl, bl, wr, y2, nw, nb, w2, b2)


def kernel(x, edge_index, lin1_w, lin1_b,
           l0g0_ln_w, l0g0_ln_b, l0g0_wl, l0g0_bl, l0g0_wr,
           l0g1_ln_w, l0g1_ln_b, l0g1_wl, l0g1_bl, l0g1_wr,
           l1g0_ln_w, l1g0_ln_b, l1g0_wl, l1g0_bl, l1g0_wr,
           l1g1_ln_w, l1g1_ln_b, l1g1_wl, l1g1_bl, l1g1_wr,
           norm_w, norm_b, lin2_w, lin2_b):
    src = edge_index[0].astype(jnp.int32)
    dst = edge_index[1].astype(jnp.int32)
    z128 = jnp.zeros((C, D), _f32)
    o128 = jnp.ones((C, D), _f32)

    r2 = lambda v: v.reshape(1, -1)

    ca, cb = _sc_deg_kernel()(dst, z128, o128)

    x0, x1, t0 = _t0_call(x, lin1_w, r2(lin1_b), r2(l0g0_ln_w), r2(l0g0_ln_b))

    pa, pb = _sc_agg(t0, src, dst, z128)
    y0, t1 = _tmid_call(x0, pa, pb, ca, cb, t0,
                        l0g0_wl, r2(l0g0_bl), l0g0_wr,
                        r2(l0g1_ln_w), r2(l0g1_ln_b))

    pa, pb = _sc_agg(t1, src, dst, z128)
    y1, t2 = _tmid_call(x1, pa, pb, ca, cb, t1,
                        l0g1_wl, r2(l0g1_bl), l0g1_wr,
                        r2(l1g0_ln_w), r2(l1g0_ln_b))

    pa, pb = _sc_agg(t2, src, dst, z128)
    y2, t3 = _tmid_call(y0, pa, pb, ca, cb, t2,
                        l1g0_wl, r2(l1g0_bl), l1g0_wr,
                        r2(l1g1_ln_w), r2(l1g1_ln_b))

    pa, pb = _sc_agg(t3, src, dst, z128)
    out = _tfin_call(y1, pa, pb, ca, cb, t3,
                     l1g1_wl, r2(l1g1_bl), l1g1_wr, y2,
                     r2(norm_w), r2(norm_b), lin2_w, r2(lin2_b))
    return out


# speedup vs baseline: 3.1119x; 1.0229x over previous
import functools

import jax
import jax.numpy as jnp
from jax import lax
from jax.experimental import pallas as pl
from jax.experimental.pallas import tpu as pltpu
from jax.experimental.pallas import tpu_sc as plsc

N = 10000
E = 320000
D = 128
DH = 256
NC = 2
NS = 16
NW = NC * NS
EPT = E // NW
C = 80
NCHUNK = EPT // C
NPAD = 10240
RPS = NPAD // NS
RB = 2000
GRID = N // RB
RW = RPS // C


def _sc_agg_body(table, srcs, dsts, z128,
                 part_a, part_b,
                 acc, sidx_a, didx_a, sidx_b, didx_b,
                 rows_a, rows_b, sg_a, sg_b, si_a, si_b):
    c = lax.axis_index("c")
    s = lax.axis_index("s")
    wid = c * NS + s
    row0 = s * RPS
    pltpu.sync_copy(z128, rows_a)
    for k in range(RW):
        pltpu.async_copy(rows_a, acc.at[pl.ds(row0 + k * C, C)], si_a)
    for k in range(RW):
        pltpu.make_async_copy(rows_a, acc.at[pl.ds(row0 + k * C, C)], si_a).wait()
    plsc.subcore_barrier()

    base = wid * EPT

    def load_idx(i, sidx, didx, sem):
        off = base + i * C
        pltpu.async_copy(srcs.at[pl.ds(off, C)], sidx, sem)
        pltpu.async_copy(dsts.at[pl.ds(off, C)], didx, sem)

    def wait_idx(i, sidx, didx, sem):
        off = base + i * C
        pltpu.make_async_copy(srcs.at[pl.ds(off, C)], sidx, sem).wait()
        pltpu.make_async_copy(dsts.at[pl.ds(off, C)], didx, sem).wait()

    load_idx(0, sidx_a, didx_a, si_a)
    wait_idx(0, sidx_a, didx_a, si_a)
    pltpu.async_copy(table.at[sidx_a], rows_a, sg_a)
    load_idx(1, sidx_b, didx_b, si_b)

    def pair(k, carry):
        i0 = 2 * k
        wait_idx(i0 + 1, sidx_b, didx_b, si_b)
        pltpu.async_copy(table.at[sidx_b], rows_b, sg_b)
        pltpu.make_async_copy(table.at[sidx_a], rows_a, sg_a).wait()
        pltpu.sync_copy(rows_a, acc.at[didx_a], add=True)
        load_idx(i0 + 2, sidx_a, didx_a, si_a)
        wait_idx(i0 + 2, sidx_a, didx_a, si_a)
        pltpu.async_copy(table.at[sidx_a], rows_a, sg_a)
        nxt = jnp.minimum(i0 + 3, NCHUNK - 1)
        load_idx(nxt, sidx_b, didx_b, si_b)
        pltpu.make_async_copy(table.at[sidx_b], rows_b, sg_b).wait()
        pltpu.sync_copy(rows_b, acc.at[didx_b], add=True)
        return carry

    lax.fori_loop(0, (NCHUNK - 1) // 2, pair, 0)
    pltpu.make_async_copy(table.at[sidx_a], rows_a, sg_a).wait()
    pltpu.sync_copy(rows_a, acc.at[didx_a], add=True)
    wait_idx(NCHUNK - 1, sidx_b, didx_b, si_b)
    plsc.subcore_barrier()

    part = [part_a, part_b]
    for ci in range(NC):
        @pl.when(c == ci)
        def _(ci=ci):
            bufs = [rows_a, rows_b]
            sems = [si_a, si_b]
            for k in range(RW):
                r0 = row0 + k * C
                buf, sem = bufs[k % 2], sems[k % 2]
                if k >= 2:
                    rp = row0 + (k - 2) * C
                    pltpu.make_async_copy(buf, part[ci].at[pl.ds(rp, C)],
                                          sem).wait()
                pltpu.sync_copy(acc.at[pl.ds(r0, C)], buf)
                pltpu.async_copy(buf, part[ci].at[pl.ds(r0, C)], sem)
            for k in range(RW - 2, RW):
                r0 = row0 + k * C
                pltpu.make_async_copy(bufs[k % 2], part[ci].at[pl.ds(r0, C)],
                                      sems[k % 2]).wait()


def _sc_deg_body(dsts, z128, o128,
                 part_a, part_b,
                 acc, didx_a, didx_b, rows_a, rows_o, si_a, si_b):
    c = lax.axis_index("c")
    s = lax.axis_index("s")
    wid = c * NS + s
    row0 = s * RPS
    pltpu.sync_copy(z128, rows_a)
    for k in range(RW):
        pltpu.async_copy(rows_a, acc.at[pl.ds(row0 + k * C, C)], si_a)
    pltpu.sync_copy(o128, rows_o)
    for k in range(RW):
        pltpu.make_async_copy(rows_a, acc.at[pl.ds(row0 + k * C, C)], si_a).wait()
    plsc.subcore_barrier()

    base = wid * EPT

    def load_idx(i, didx, sem):
        pltpu.async_copy(dsts.at[pl.ds(base + i * C, C)], didx, sem)

    def wait_idx(i, didx, sem):
        pltpu.make_async_copy(dsts.at[pl.ds(base + i * C, C)], didx, sem).wait()

    load_idx(0, didx_a, si_a)
    wait_idx(0, didx_a, si_a)
    load_idx(1, didx_b, si_b)

    def pair(k, carry):
        i0 = 2 * k
        pltpu.sync_copy(rows_o, acc.at[didx_a], add=True)
        load_idx(i0 + 2, didx_a, si_a)
        wait_idx(i0 + 1, didx_b, si_b)
        pltpu.sync_copy(rows_o, acc.at[didx_b], add=True)
        load_idx(jnp.minimum(i0 + 3, NCHUNK - 1), didx_b, si_b)
        wait_idx(i0 + 2, didx_a, si_a)
        return carry

    lax.fori_loop(0, (NCHUNK - 1) // 2, pair, 0)
    pltpu.sync_copy(rows_o, acc.at[didx_a], add=True)
    wait_idx(NCHUNK - 1, didx_b, si_b)
    plsc.subcore_barrier()

    part = [part_a, part_b]
    for ci in range(NC):
        @pl.when(c == ci)
        def _(ci=ci):
            bufs = [rows_a, rows_o]
            sems = [si_a, si_b]
            for k in range(RW):
                r0 = row0 + k * C
                buf, sem = bufs[k % 2], sems[k % 2]
                if k >= 2:
                    rp = row0 + (k - 2) * C
                    pltpu.make_async_copy(buf, part[ci].at[pl.ds(rp, C)],
                                          sem).wait()
                pltpu.sync_copy(acc.at[pl.ds(r0, C)], buf)
                pltpu.async_copy(buf, part[ci].at[pl.ds(r0, C)], sem)
            for k in range(RW - 2, RW):
                r0 = row0 + k * C
                pltpu.make_async_copy(bufs[k % 2], part[ci].at[pl.ds(r0, C)],
                                      sems[k % 2]).wait()


@functools.cache
def _sc_deg_kernel():
    mesh = plsc.VectorSubcoreMesh(core_axis_name="c", subcore_axis_name="s")
    return pl.kernel(
        _sc_deg_body,
        out_type=(
            jax.ShapeDtypeStruct((NPAD, D), jnp.float32),
            jax.ShapeDtypeStruct((NPAD, D), jnp.float32),
        ),
        mesh=mesh,
        scratch_types=[
            pltpu.VMEM_SHARED((NPAD, D), jnp.float32),
            pltpu.VMEM((C,), jnp.int32),
            pltpu.VMEM((C,), jnp.int32),
            pltpu.VMEM((C, D), jnp.float32),
            pltpu.VMEM((C, D), jnp.float32),
            pltpu.SemaphoreType.DMA,
            pltpu.SemaphoreType.DMA,
        ],
        name="sc_degree_segsum",
    )


@functools.cache
def _sc_kernel():
    mesh = plsc.VectorSubcoreMesh(core_axis_name="c", subcore_axis_name="s")
    return pl.kernel(
        _sc_agg_body,
        out_type=(
            jax.ShapeDtypeStruct((NPAD, D), jnp.float32),
            jax.ShapeDtypeStruct((NPAD, D), jnp.float32),
        ),
        mesh=mesh,
        scratch_types=[
            pltpu.VMEM_SHARED((NPAD, D), jnp.float32),
            pltpu.VMEM((C,), jnp.int32),
            pltpu.VMEM((C,), jnp.int32),
            pltpu.VMEM((C,), jnp.int32),
            pltpu.VMEM((C,), jnp.int32),
            pltpu.VMEM((C, D), jnp.float32),
            pltpu.VMEM((C, D), jnp.float32),
            pltpu.SemaphoreType.DMA,
            pltpu.SemaphoreType.DMA,
            pltpu.SemaphoreType.DMA,
            pltpu.SemaphoreType.DMA,
        ],
        name="sc_gather_segsum",
    )


def _sc_agg(*args):
    return _sc_kernel()(*args)


def _relu_ln(v, w, b):
    mu = jnp.mean(v, axis=-1, keepdims=True)
    var = jnp.mean((v - mu) ** 2, axis=-1, keepdims=True)
    return jax.nn.relu((v - mu) * lax.rsqrt(var + 1e-5) * w + b)


def _dot_t(a, w):
    return lax.dot_general(a, w, (((1,), (1,)), ((), ())),
                           preferred_element_type=jnp.float32)


def _t0_body(x_ref, w1_ref, b1_ref, lnw_ref, lnb_ref, x0_ref, x1_ref, t0_ref):
    h = _dot_t(x_ref[...], w1_ref[...]) + b1_ref[...]
    x0 = h[:, :D]
    x1 = h[:, D:]
    x0_ref[...] = x0
    x1_ref[...] = x1
    t0_ref[...] = _relu_ln(x1, lnw_ref[...], lnb_ref[...])


def _tmid_body(res_ref, pa_ref, pb_ref, ca_ref, cb_ref, tab_ref,
               wl_ref, bl_ref, wr_ref, lnw_ref, lnb_ref, y_ref, tn_ref):
    cnt = ca_ref[:, 0] + cb_ref[:, 0]
    inv = 1.0 / jnp.maximum(cnt, 1.0)
    agg = (pa_ref[...] + pb_ref[...]) * inv[:, None]
    y = (res_ref[...] + _dot_t(agg, wl_ref[...]) + bl_ref[...]
         + _dot_t(tab_ref[...], wr_ref[...]))
    y_ref[...] = y
    tn_ref[...] = _relu_ln(y, lnw_ref[...], lnb_ref[...])


def _tfin_body(res_ref, pa_ref, pb_ref, ca_ref, cb_ref, tab_ref,
               wl_ref, bl_ref, wr_ref, y2_ref, nw_ref, nb_ref,
               w2_ref, b2_ref, out_ref):
    cnt = ca_ref[:, 0] + cb_ref[:, 0]
    inv = 1.0 / jnp.maximum(cnt, 1.0)
    agg = (pa_ref[...] + pb_ref[...]) * inv[:, None]
    y3 = (res_ref[...] + _dot_t(agg, wl_ref[...]) + bl_ref[...]
          + _dot_t(tab_ref[...], wr_ref[...]))
    h = jnp.concatenate([y2_ref[...], y3], axis=1)
    hn = _relu_ln(h, nw_ref[...], nb_ref[...])
    out_ref[...] = _dot_t(hn, w2_ref[...]) + b2_ref[...]


def _rows(shape):
    return pl.BlockSpec((RB,) + shape[1:], lambda i: (i,) + (0,) * (len(shape) - 1))


def _full(shape):
    return pl.BlockSpec(shape, lambda i: (0,) * len(shape))


_f32 = jnp.float32


def _t0_call(x, w1, b1, lnw, lnb):
    return pl.pallas_call(
        _t0_body,
        grid=(GRID,),
        in_specs=[_rows((N, D)), _full((DH, D)), _full((1, DH)),
                  _full((1, D)), _full((1, D))],
        out_specs=(_rows((N, D)), _rows((N, D)), _rows((N, D))),
        out_shape=(jax.ShapeDtypeStruct((N, D), _f32),) * 3,
        name="tc_lin1_ln",
    )(x, w1, b1, lnw, lnb)


def _tmid_call(res, pa, pb, ca, cb, tab, wl, bl, wr, lnw, lnb):
    return pl.pallas_call(
        _tmid_body,
        grid=(GRID,),
        in_specs=[_rows((N, D)), _rows((N, D)), _rows((N, D)),
                  _rows((N, D)), _rows((N, D)), _rows((N, D)),
                  _full((D, D)), _full((1, D)), _full((D, D)),
                  _full((1, D)), _full((1, D))],
        out_specs=(_rows((N, D)), _rows((N, D))),
        out_shape=(jax.ShapeDtypeStruct((N, D), _f32),) * 2,
        name="tc_block_post",
    )(res, pa, pb, ca, cb, tab, wl, bl, wr, lnw, lnb)


def _tfin_call(res, pa, pb, ca, cb, tab, wl, bl, wr, y2, nw, nb, w2, b2):
    return pl.pallas_call(
        _tfin_body,
        grid=(GRID,),
        in_specs=[_rows((N, D)), _rows((N, D)), _rows((N, D)),
                  _rows((N, D)), _rows((N, D)), _rows((N, D)),
                  _full((D, D)), _full((1, D)), _full((D, D)),
                  _rows((N, D)), _full((1, DH)), _full((1, DH)),
                  _full((D, DH)), _full((1, D))],
        out_specs=_rows((N, D)),
        out_shape=jax.ShapeDtypeStruct((N, D), _f32),
        name="tc_final",
    )(res, pa, pb, ca, cb, tab, wl, bl, wr, y2, nw, nb, w2, b2)


def kernel(x, edge_index, lin1_w, lin1_b,
           l0g0_ln_w, l0g0_ln_b, l0g0_wl, l0g0_bl, l0g0_wr,
           l0g1_ln_w, l0g1_ln_b, l0g1_wl, l0g1_bl, l0g1_wr,
           l1g0_ln_w, l1g0_ln_b, l1g0_wl, l1g0_bl, l1g0_wr,
           l1g1_ln_w, l1g1_ln_b, l1g1_wl, l1g1_bl, l1g1_wr,
           norm_w, norm_b, lin2_w, lin2_b):
    src = edge_index[0].astype(jnp.int32)
    dst = edge_index[1].astype(jnp.int32)
    z128 = jnp.zeros((C, D), _f32)
    o128 = jnp.ones((C, D), _f32)

    r2 = lambda v: v.reshape(1, -1)

    ca, cb = _sc_deg_kernel()(dst, z128, o128)

    x0, x1, t0 = _t0_call(x, lin1_w, r2(lin1_b), r2(l0g0_ln_w), r2(l0g0_ln_b))

    pa, pb = _sc_agg(t0, src, dst, z128)
    y0, t1 = _tmid_call(x0, pa, pb, ca, cb, t0,
                        l0g0_wl, r2(l0g0_bl), l0g0_wr,
                        r2(l0g1_ln_w), r2(l0g1_ln_b))

    pa, pb = _sc_agg(t1, src, dst, z128)
    y1, t2 = _tmid_call(x1, pa, pb, ca, cb, t1,
                        l0g1_wl, r2(l0g1_bl), l0g1_wr,
                        r2(l1g0_ln_w), r2(l1g0_ln_b))

    pa, pb = _sc_agg(t2, src, dst, z128)
    y2, t3 = _tmid_call(y0, pa, pb, ca, cb, t2,
                        l1g0_wl, r2(l1g0_bl), l1g0_wr,
                        r2(l1g1_ln_w), r2(l1g1_ln_b))

    pa, pb = _sc_agg(t3, src, dst, z128)
    out = _tfin_call(y1, pa, pb, ca, cb, t3,
                     l1g1_wl, r2(l1g1_bl), l1g1_wr, y2,
                     r2(norm_w), r2(norm_b), lin2_w, r2(lin2_b))
    return out
